# Initial kernel scaffold; baseline (speedup 1.0000x reference)
import jax
import jax.numpy as jnp
from jax.experimental import pallas as pl


def kernel(x, edge_index, W1, b1, W2, b2):
    raise NotImplementedError("write your pallas kernel here")


# speedup vs baseline: 9.8852x; 9.8852x over previous
import functools

import jax
import jax.numpy as jnp
from jax import lax
from jax.experimental import pallas as pl
from jax.experimental.pallas import tpu as pltpu
from jax.experimental.pallas import tpu_sc as plsc

NC = 2
NS = 16
NW = NC * NS
K = 80
NBUF = 2


def _mesh():
    return plsc.VectorSubcoreMesh(
        core_axis_name="c", subcore_axis_name="s", num_cores=NC, num_subcores=NS
    )


@functools.cache
def _make_deg_kernel(n, nch):
    rpt = (n // NS) & ~7
    tail = n - NS * rpt

    def body(src_hbm, dst_hbm, ones_hbm, zeros_hbm, out_hbm,
             si0, si1, di0, di1, ones_v, stage,
             ssem0, ssem1, dsem0, dsem1,
             sa0, sa1, sb0, sb1, deg_o, deg_i):
        si = (si0, si1)
        di = (di0, di1)
        ssem = (ssem0, ssem1)
        dsem = (dsem0, dsem1)
        sa = (sa0, sa1)
        sb = (sb0, sb1)
        c = lax.axis_index("c")
        s = lax.axis_index("s")
        wid = c * NS + s
        base = pl.multiple_of(wid * (nch * K), K)
        pltpu.sync_copy(ones_hbm, ones_v)

        def load(j, b):
            off = base + pl.multiple_of(j * K, K)
            pltpu.async_copy(src_hbm.at[pl.ds(off, K)], si[b], ssem[b])
            pltpu.async_copy(dst_hbm.at[pl.ds(off, K)], di[b], dsem[b])

        def wait_load(j, b):
            off = base + pl.multiple_of(j * K, K)
            pltpu.make_async_copy(src_hbm.at[pl.ds(off, K)], si[b], ssem[b]).wait()
            pltpu.make_async_copy(dst_hbm.at[pl.ds(off, K)], di[b], dsem[b]).wait()

        def fire_scat(b):
            pltpu.async_copy(ones_v, deg_o.at[si[b]], sa[b], add=True)
            pltpu.async_copy(ones_v, deg_i.at[di[b]], sb[b], add=True)

        def drain_scat(b):
            pltpu.make_async_copy(ones_v, deg_o.at[si[b]], sa[b]).wait()
            pltpu.make_async_copy(ones_v, deg_i.at[di[b]], sb[b]).wait()

        load(0, 0)
        pltpu.sync_copy(zeros_hbm, stage)
        pltpu.sync_copy(stage.at[pl.ds(0, rpt)], deg_o.at[pl.ds(s * rpt, rpt)])
        pltpu.sync_copy(stage.at[pl.ds(0, rpt)], deg_i.at[pl.ds(s * rpt, rpt)])

        @pl.when(s == NS - 1)
        def _():
            pltpu.sync_copy(stage.at[pl.ds(0, tail)],
                            deg_o.at[pl.ds(NS * rpt, tail)])
            pltpu.sync_copy(stage.at[pl.ds(0, tail)],
                            deg_i.at[pl.ds(NS * rpt, tail)])

        plsc.subcore_barrier()
        wait_load(0, 0)
        fire_scat(0)
        load(1, 1)

        main = ((nch - 2) // 2) * 2
        if main >= 2:
            @pl.loop(1, main + 1, step=2)
            def _(jo):
                for uo in range(2):
                    j = jo + uo
                    u = (1 + uo) % 2
                    wait_load(j, u)
                    fire_scat(u)
                    drain_scat(1 - u)
                    load(j + 1, 1 - u)

        for j in range(main + 1, nch):
            u = j % 2
            wait_load(j, u)
            fire_scat(u)
            drain_scat(1 - u)
            if j + 1 < nch:
                load(j + 1, 1 - u)

        drain_scat((nch - 1) % 2)
        plsc.subcore_barrier()

        for t, ref in ((0, deg_o), (1, deg_i)):
            obase = pl.multiple_of((c * 2 + t) * n, 8)
            pltpu.sync_copy(ref.at[pl.ds(s * rpt, rpt)], stage.at[pl.ds(0, rpt)])
            pltpu.sync_copy(stage.at[pl.ds(0, rpt)],
                            out_hbm.at[pl.ds(obase + s * rpt, rpt)])

            @pl.when(s == NS - 1)
            def _():
                pltpu.sync_copy(ref.at[pl.ds(NS * rpt, tail)],
                                stage.at[pl.ds(rpt, tail)])
                pltpu.sync_copy(stage.at[pl.ds(rpt, tail)],
                                out_hbm.at[pl.ds(obase + NS * rpt, tail)])

    return pl.kernel(
        body,
        out_type=jax.ShapeDtypeStruct((NC * 2 * n,), jnp.float32),
        mesh=_mesh(),
        scratch_types=[
            pltpu.VMEM((K,), jnp.int32),
            pltpu.VMEM((K,), jnp.int32),
            pltpu.VMEM((K,), jnp.int32),
            pltpu.VMEM((K,), jnp.int32),
            pltpu.VMEM((K,), jnp.float32),
            pltpu.VMEM((rpt + tail,), jnp.float32),
            pltpu.SemaphoreType.DMA,
            pltpu.SemaphoreType.DMA,
            pltpu.SemaphoreType.DMA,
            pltpu.SemaphoreType.DMA,
            pltpu.SemaphoreType.DMA,
            pltpu.SemaphoreType.DMA,
            pltpu.SemaphoreType.DMA,
            pltpu.SemaphoreType.DMA,
            pltpu.VMEM_SHARED((n,), jnp.float32),
            pltpu.VMEM_SHARED((n,), jnp.float32),
        ],
        name="gcn_degrees_sc",
    )


@functools.cache
def _make_agg_kernel(n, d, nch):
    rpt = n // NS

    nb = 3

    def body(y_hbm, src_hbm, dst_hbm, zeros_hbm, out_hbm,
             si0, si1, si2, di0, di1, di2, rows,
             ss0, ss1, ss2, ds0, ds1, ds2, gs0, gs1, gs2, acc):
        si = (si0, si1, si2)
        di = (di0, di1, di2)
        ssem = (ss0, ss1, ss2)
        dsem = (ds0, ds1, ds2)
        gsem = (gs0, gs1, gs2)
        c = lax.axis_index("c")
        s = lax.axis_index("s")
        wid = c * NS + s
        base = pl.multiple_of(wid * (nch * K), K)

        def load(j, b):
            off = base + pl.multiple_of(j * K, K)
            pltpu.async_copy(src_hbm.at[pl.ds(off, K)], si[b], ssem[b])
            pltpu.async_copy(dst_hbm.at[pl.ds(off, K)], di[b], dsem[b])

        def gather(j, b):
            off = base + pl.multiple_of(j * K, K)
            pltpu.make_async_copy(src_hbm.at[pl.ds(off, K)], si[b], ssem[b]).wait()
            pltpu.async_copy(y_hbm.at[si[b]], rows.at[b], gsem[b])

        def scat(j, b):
            off = base + pl.multiple_of(j * K, K)
            pltpu.make_async_copy(y_hbm.at[si[b]], rows.at[b], gsem[b]).wait()
            pltpu.make_async_copy(dst_hbm.at[pl.ds(off, K)], di[b], dsem[b]).wait()
            pltpu.sync_copy(rows.at[b], acc.at[di[b]], add=True)

        load(0, 0)
        load(1, 1)
        gather(0, 0)
        pltpu.sync_copy(zeros_hbm, acc.at[pl.ds(s * rpt, rpt)])
        plsc.subcore_barrier()

        main = nch - (nch % nb) - nb
        if main > 0:
            @pl.loop(0, main, step=nb)
            def _(jo):
                for u in range(nb):
                    j = jo + u
                    load(j + 2, (u + 2) % nb)
                    gather(j + 1, (u + 1) % nb)
                    scat(j, u)

        for j in range(max(main, 0), nch):
            if j + 2 < nch:
                load(j + 2, (j + 2) % nb)
            if j + 1 < nch:
                gather(j + 1, (j + 1) % nb)
            scat(j, j % nb)

        plsc.subcore_barrier()
        pltpu.sync_copy(acc.at[pl.ds(s * rpt, rpt)], out_hbm.at[c, s])

    return pl.kernel(
        body,
        out_type=jax.ShapeDtypeStruct((NC, NS, n // NS, d), jnp.float32),
        mesh=_mesh(),
        scratch_types=[
            pltpu.VMEM((K,), jnp.int32),
            pltpu.VMEM((K,), jnp.int32),
            pltpu.VMEM((K,), jnp.int32),
            pltpu.VMEM((K,), jnp.int32),
            pltpu.VMEM((K,), jnp.int32),
            pltpu.VMEM((K,), jnp.int32),
            pltpu.VMEM((nb, K, d), jnp.float32),
            pltpu.SemaphoreType.DMA,
            pltpu.SemaphoreType.DMA,
            pltpu.SemaphoreType.DMA,
            pltpu.SemaphoreType.DMA,
            pltpu.SemaphoreType.DMA,
            pltpu.SemaphoreType.DMA,
            pltpu.SemaphoreType.DMA,
            pltpu.SemaphoreType.DMA,
            pltpu.SemaphoreType.DMA,
            pltpu.VMEM_SHARED((n, d), jnp.float32),
        ],
        name="gcn_edge_agg_sc",
    )


def _scale_body(deg_ref, x_ref, y_ref, rso_ref, rsi_ref):
    d = deg_ref[...]
    rso = lax.rsqrt(jnp.maximum(d[:, 0:1] + d[:, 2:3], 1.0))
    rsi = lax.rsqrt(jnp.maximum(d[:, 1:2] + d[:, 3:4], 1.0))
    rso_ref[...] = rso
    rsi_ref[...] = rsi
    y_ref[...] = x_ref[...] * rso


@functools.cache
def _make_scale_call(n, d, r):
    return pl.pallas_call(
        _scale_body,
        grid=(n // r,),
        in_specs=[
            pl.BlockSpec((r, 4), lambda i: (i, 0)),
            pl.BlockSpec((r, d), lambda i: (i, 0)),
        ],
        out_specs=[
            pl.BlockSpec((r, d), lambda i: (i, 0)),
            pl.BlockSpec((r, 1), lambda i: (i, 0)),
            pl.BlockSpec((r, 1), lambda i: (i, 0)),
        ],
        out_shape=[
            jax.ShapeDtypeStruct((n, d), jnp.float32),
            jax.ShapeDtypeStruct((n, 1), jnp.float32),
            jax.ShapeDtypeStruct((n, 1), jnp.float32),
        ],
        name="gcn_scale_tc",
    )


def _mid_body(p_ref, rso_ref, rsi_ref, w1_ref, b1_ref, w2_ref, y2_ref):
    agg = (p_ref[0] + p_ref[1]) * rsi_ref[...]
    h1 = jnp.dot(agg, w1_ref[...], preferred_element_type=jnp.float32,
                 precision=lax.Precision.HIGHEST) + b1_ref[...]
    h1 = jnp.maximum(h1, 0.0)
    y2_ref[...] = jnp.dot(h1 * rso_ref[...], w2_ref[...],
                          preferred_element_type=jnp.float32,
                          precision=lax.Precision.HIGHEST)


@functools.cache
def _make_mid_call(n, d, dh, do, r):
    return pl.pallas_call(
        _mid_body,
        grid=(n // r,),
        in_specs=[
            pl.BlockSpec((NC, r, d), lambda i: (0, i, 0)),
            pl.BlockSpec((r, 1), lambda i: (i, 0)),
            pl.BlockSpec((r, 1), lambda i: (i, 0)),
            pl.BlockSpec((d, dh), lambda i: (0, 0)),
            pl.BlockSpec((1, dh), lambda i: (0, 0)),
            pl.BlockSpec((dh, do), lambda i: (0, 0)),
        ],
        out_specs=pl.BlockSpec((r, do), lambda i: (i, 0)),
        out_shape=jax.ShapeDtypeStruct((n, do), jnp.float32),
        name="gcn_mid_tc",
    )


def _final_body(p_ref, rsi_ref, b2_ref, h2_ref):
    agg = (p_ref[0] + p_ref[1]) * rsi_ref[...]
    h2_ref[...] = jnp.maximum(agg + b2_ref[...], 0.0)


@functools.cache
def _make_final_call(n, do, r):
    return pl.pallas_call(
        _final_body,
        grid=(n // r,),
        in_specs=[
            pl.BlockSpec((NC, r, do), lambda i: (0, i, 0)),
            pl.BlockSpec((r, 1), lambda i: (i, 0)),
            pl.BlockSpec((1, do), lambda i: (0, 0)),
        ],
        out_specs=pl.BlockSpec((r, do), lambda i: (i, 0)),
        out_shape=jax.ShapeDtypeStruct((n, do), jnp.float32),
        name="gcn_final_tc",
    )


def kernel(x, edge_index, W1, b1, W2, b2):
    n, d_in = x.shape
    e = edge_index.shape[1]
    d_hid = W1.shape[1]
    d_out = W2.shape[1]
    ew = e // NW
    nch = ew // K
    r = 1000

    ones1 = jnp.ones((K,), jnp.float32)
    rpt8 = (n // NS) & ~7
    zeros1 = jnp.zeros((rpt8 + (n - NS * rpt8),), jnp.float32)
    zeros_d = jnp.zeros((n // NS, d_in), jnp.float32)

    src_flat = edge_index[0]
    dst_flat = edge_index[1]
    deg = _make_deg_kernel(n, nch)(src_flat, dst_flat, ones1, zeros1)
    deg4 = deg.reshape(NC * 2, n).T
    y1, rso, rsi = _make_scale_call(n, d_in, r)(deg4, x)
    p1 = _make_agg_kernel(n, d_in, nch)(y1, src_flat, dst_flat, zeros_d)
    y2 = _make_mid_call(n, d_in, d_hid, d_out, r)(
        p1.reshape(NC, n, d_in), rso, rsi, W1, b1.reshape(1, -1), W2)
    p2 = _make_agg_kernel(n, d_out, nch)(y2, src_flat, dst_flat, zeros_d)
    h2 = _make_final_call(n, d_out, r)(
        p2.reshape(NC, n, d_out), rsi, b2.reshape(1, -1))
    return h2


# speedup vs baseline: 11.0197x; 1.1148x over previous
import functools

import jax
import jax.numpy as jnp
from jax import lax
from jax.experimental import pallas as pl
from jax.experimental.pallas import tpu as pltpu
from jax.experimental.pallas import tpu_sc as plsc

NC = 2
NS = 16
NW = NC * NS
K = 80
NBUF = 2


def _mesh():
    return plsc.VectorSubcoreMesh(
        core_axis_name="c", subcore_axis_name="s", num_cores=NC, num_subcores=NS
    )


@functools.cache
def _make_deg_kernel(n, ew, eoff):
    rpt = (n // NS) & ~7
    tail = n - NS * rpt
    kd = 128
    nch = ew // kd
    kt = ew - nch * kd

    def body(edges_hbm, ones_hbm, zeros_hbm, out_hbm,
             si0, si1, di0, di1, si_t, di_t, ones_v, ones_t, stage,
             ssem0, ssem1, dsem0, dsem1,
             sa0, sa1, sb0, sb1, deg_o, deg_i):
        si = (si0, si1)
        di = (di0, di1)
        ssem = (ssem0, ssem1)
        dsem = (dsem0, dsem1)
        sa = (sa0, sa1)
        sb = (sb0, sb1)
        c = lax.axis_index("c")
        s = lax.axis_index("s")
        wid = c * NS + s
        base = pl.multiple_of(wid * ew, 8)
        pltpu.sync_copy(ones_hbm, ones_v)

        def load(j, b):
            off = base + pl.multiple_of(j * kd, 8)
            pltpu.async_copy(edges_hbm.at[pl.ds(off, kd)], si[b], ssem[b])
            pltpu.async_copy(edges_hbm.at[pl.ds(eoff + off, kd)], di[b], dsem[b])

        def wait_load(j, b):
            off = base + pl.multiple_of(j * kd, 8)
            pltpu.make_async_copy(edges_hbm.at[pl.ds(off, kd)], si[b], ssem[b]).wait()
            pltpu.make_async_copy(edges_hbm.at[pl.ds(eoff + off, kd)], di[b], dsem[b]).wait()

        def fire_scat(b):
            pltpu.async_copy(ones_v, deg_o.at[si[b]], sa[b], add=True)
            pltpu.async_copy(ones_v, deg_i.at[di[b]], sb[b], add=True)

        def drain_scat(b):
            pltpu.make_async_copy(ones_v, deg_o.at[si[b]], sa[b]).wait()
            pltpu.make_async_copy(ones_v, deg_i.at[di[b]], sb[b]).wait()

        load(0, 0)
        pltpu.sync_copy(zeros_hbm, stage)
        pltpu.sync_copy(stage.at[pl.ds(0, rpt)], deg_o.at[pl.ds(s * rpt, rpt)])
        pltpu.sync_copy(stage.at[pl.ds(0, rpt)], deg_i.at[pl.ds(s * rpt, rpt)])

        @pl.when(s == NS - 1)
        def _():
            pltpu.sync_copy(stage.at[pl.ds(0, tail)],
                            deg_o.at[pl.ds(NS * rpt, tail)])
            pltpu.sync_copy(stage.at[pl.ds(0, tail)],
                            deg_i.at[pl.ds(NS * rpt, tail)])

        plsc.subcore_barrier()
        wait_load(0, 0)
        fire_scat(0)
        load(1, 1)

        main = ((nch - 2) // 2) * 2
        if main >= 2:
            @pl.loop(1, main + 1, step=2)
            def _(jo):
                for uo in range(2):
                    j = jo + uo
                    u = (1 + uo) % 2
                    wait_load(j, u)
                    fire_scat(u)
                    drain_scat(1 - u)
                    load(j + 1, 1 - u)

        for j in range(main + 1, nch):
            u = j % 2
            wait_load(j, u)
            fire_scat(u)
            drain_scat(1 - u)
            if j + 1 < nch:
                load(j + 1, 1 - u)

        drain_scat((nch - 1) % 2)

        if kt:
            toff = base + pl.multiple_of(nch * kd, 8)
            pltpu.sync_copy(edges_hbm.at[pl.ds(toff, kt)], si_t)
            pltpu.sync_copy(edges_hbm.at[pl.ds(eoff + toff, kt)], di_t)
            pltpu.sync_copy(ones_hbm.at[pl.ds(0, kt)], ones_t)
            pltpu.sync_copy(ones_t, deg_o.at[si_t], add=True)
            pltpu.sync_copy(ones_t, deg_i.at[di_t], add=True)

        plsc.subcore_barrier()

        for t, ref in ((0, deg_o), (1, deg_i)):
            obase = pl.multiple_of((c * 2 + t) * n, 8)
            pltpu.sync_copy(ref.at[pl.ds(s * rpt, rpt)], stage.at[pl.ds(0, rpt)])
            pltpu.sync_copy(stage.at[pl.ds(0, rpt)],
                            out_hbm.at[pl.ds(obase + s * rpt, rpt)])

            @pl.when(s == NS - 1)
            def _():
                pltpu.sync_copy(ref.at[pl.ds(NS * rpt, tail)],
                                stage.at[pl.ds(rpt, tail)])
                pltpu.sync_copy(stage.at[pl.ds(rpt, tail)],
                                out_hbm.at[pl.ds(obase + NS * rpt, tail)])

    return pl.kernel(
        body,
        out_type=jax.ShapeDtypeStruct((NC * 2 * n,), jnp.float32),
        mesh=_mesh(),
        scratch_types=[
            pltpu.VMEM((kd,), jnp.int32),
            pltpu.VMEM((kd,), jnp.int32),
            pltpu.VMEM((kd,), jnp.int32),
            pltpu.VMEM((kd,), jnp.int32),
            pltpu.VMEM((max(kt, 8),), jnp.int32),
            pltpu.VMEM((max(kt, 8),), jnp.int32),
            pltpu.VMEM((kd,), jnp.float32),
            pltpu.VMEM((max(kt, 8),), jnp.float32),
            pltpu.VMEM((rpt + tail,), jnp.float32),
            pltpu.SemaphoreType.DMA,
            pltpu.SemaphoreType.DMA,
            pltpu.SemaphoreType.DMA,
            pltpu.SemaphoreType.DMA,
            pltpu.SemaphoreType.DMA,
            pltpu.SemaphoreType.DMA,
            pltpu.SemaphoreType.DMA,
            pltpu.SemaphoreType.DMA,
            pltpu.VMEM_SHARED((n,), jnp.float32),
            pltpu.VMEM_SHARED((n,), jnp.float32),
        ],
        name="gcn_degrees_sc",
    )


@functools.cache
def _make_agg_kernel(n, d, ew, eoff):
    rpt = (n // NS) & ~7
    tailr = n - NS * rpt
    nch = ew // K

    nb = 3

    def body(y_hbm, edges_hbm, zeros_hbm, out_hbm,
             si0, si1, si2, di0, di1, di2, rows,
             ss0, ss1, ss2, ds0, ds1, ds2, gs0, gs1, gs2, acc):
        si = (si0, si1, si2)
        di = (di0, di1, di2)
        ssem = (ss0, ss1, ss2)
        dsem = (ds0, ds1, ds2)
        gsem = (gs0, gs1, gs2)
        c = lax.axis_index("c")
        s = lax.axis_index("s")
        wid = c * NS + s
        base = pl.multiple_of(wid * ew, 8)

        def load(j, b):
            off = base + pl.multiple_of(j * K, K)
            pltpu.async_copy(edges_hbm.at[pl.ds(off, K)], si[b], ssem[b])
            pltpu.async_copy(edges_hbm.at[pl.ds(eoff + off, K)], di[b], dsem[b])

        def gather(j, b):
            off = base + pl.multiple_of(j * K, K)
            pltpu.make_async_copy(edges_hbm.at[pl.ds(off, K)], si[b], ssem[b]).wait()
            pltpu.async_copy(y_hbm.at[si[b]], rows.at[b], gsem[b])

        def scat(j, b):
            off = base + pl.multiple_of(j * K, K)
            pltpu.make_async_copy(y_hbm.at[si[b]], rows.at[b], gsem[b]).wait()
            pltpu.make_async_copy(edges_hbm.at[pl.ds(eoff + off, K)], di[b], dsem[b]).wait()
            pltpu.sync_copy(rows.at[b], acc.at[di[b]], add=True)

        load(0, 0)
        load(1, 1)
        gather(0, 0)
        pltpu.sync_copy(zeros_hbm.at[pl.ds(0, rpt)], acc.at[pl.ds(s * rpt, rpt)])

        @pl.when(s == NS - 1)
        def _():
            pltpu.sync_copy(zeros_hbm.at[pl.ds(0, tailr)],
                            acc.at[pl.ds(NS * rpt, tailr)])

        plsc.subcore_barrier()

        main = nch - (nch % nb) - nb
        if main > 0:
            @pl.loop(0, main, step=nb)
            def _(jo):
                for u in range(nb):
                    j = jo + u
                    load(j + 2, (u + 2) % nb)
                    gather(j + 1, (u + 1) % nb)
                    scat(j, u)

        for j in range(max(main, 0), nch):
            if j + 2 < nch:
                load(j + 2, (j + 2) % nb)
            if j + 1 < nch:
                gather(j + 1, (j + 1) % nb)
            scat(j, j % nb)

        plsc.subcore_barrier()
        pltpu.sync_copy(acc.at[pl.ds(s * rpt, rpt)],
                        out_hbm.at[c, pl.ds(s * rpt, rpt)])

        @pl.when(s == NS - 1)
        def _():
            pltpu.sync_copy(acc.at[pl.ds(NS * rpt, tailr)],
                            out_hbm.at[c, pl.ds(NS * rpt, tailr)])

    return pl.kernel(
        body,
        out_type=jax.ShapeDtypeStruct((NC, n, d), jnp.float32),
        mesh=_mesh(),
        scratch_types=[
            pltpu.VMEM((K,), jnp.int32),
            pltpu.VMEM((K,), jnp.int32),
            pltpu.VMEM((K,), jnp.int32),
            pltpu.VMEM((K,), jnp.int32),
            pltpu.VMEM((K,), jnp.int32),
            pltpu.VMEM((K,), jnp.int32),
            pltpu.VMEM((nb, K, d), jnp.float32),
            pltpu.SemaphoreType.DMA,
            pltpu.SemaphoreType.DMA,
            pltpu.SemaphoreType.DMA,
            pltpu.SemaphoreType.DMA,
            pltpu.SemaphoreType.DMA,
            pltpu.SemaphoreType.DMA,
            pltpu.SemaphoreType.DMA,
            pltpu.SemaphoreType.DMA,
            pltpu.SemaphoreType.DMA,
            pltpu.VMEM_SHARED((n, d), jnp.float32),
        ],
        name="gcn_edge_agg_sc",
    )


def _scale_body(deg_ref, x_ref, y_ref, rso_ref, rsi_ref):
    d = deg_ref[...]
    rso = lax.rsqrt(jnp.maximum(d[:, 0:1] + d[:, 2:3], 1.0))
    rsi = lax.rsqrt(jnp.maximum(d[:, 1:2] + d[:, 3:4], 1.0))
    rso_ref[...] = rso
    rsi_ref[...] = rsi
    y_ref[...] = x_ref[...] * rso


@functools.cache
def _make_scale_call(n, d, r):
    return pl.pallas_call(
        _scale_body,
        grid=(n // r,),
        in_specs=[
            pl.BlockSpec((r, 4), lambda i: (i, 0)),
            pl.BlockSpec((r, d), lambda i: (i, 0)),
        ],
        out_specs=[
            pl.BlockSpec((r, d), lambda i: (i, 0)),
            pl.BlockSpec((r, 1), lambda i: (i, 0)),
            pl.BlockSpec((r, 1), lambda i: (i, 0)),
        ],
        out_shape=[
            jax.ShapeDtypeStruct((n, d), jnp.float32),
            jax.ShapeDtypeStruct((n, 1), jnp.float32),
            jax.ShapeDtypeStruct((n, 1), jnp.float32),
        ],
        name="gcn_scale_tc",
    )


def _mid_body(p_ref, rso_ref, rsi_ref, w1_ref, b1_ref, w2_ref, y2_ref):
    agg = (p_ref[0] + p_ref[1]) * rsi_ref[...]
    h1 = jnp.dot(agg, w1_ref[...], preferred_element_type=jnp.float32,
                 precision=lax.Precision.HIGHEST) + b1_ref[...]
    h1 = jnp.maximum(h1, 0.0)
    y2_ref[...] = jnp.dot(h1 * rso_ref[...], w2_ref[...],
                          preferred_element_type=jnp.float32,
                          precision=lax.Precision.HIGHEST)


@functools.cache
def _make_mid_call(n, d, dh, do, r):
    return pl.pallas_call(
        _mid_body,
        grid=(n // r,),
        in_specs=[
            pl.BlockSpec((NC, r, d), lambda i: (0, i, 0)),
            pl.BlockSpec((r, 1), lambda i: (i, 0)),
            pl.BlockSpec((r, 1), lambda i: (i, 0)),
            pl.BlockSpec((d, dh), lambda i: (0, 0)),
            pl.BlockSpec((1, dh), lambda i: (0, 0)),
            pl.BlockSpec((dh, do), lambda i: (0, 0)),
        ],
        out_specs=pl.BlockSpec((r, do), lambda i: (i, 0)),
        out_shape=jax.ShapeDtypeStruct((n, do), jnp.float32),
        name="gcn_mid_tc",
    )


def _final_body(p_ref, rsi_ref, b2_ref, h2_ref):
    agg = (p_ref[0] + p_ref[1]) * rsi_ref[...]
    h2_ref[...] = jnp.maximum(agg + b2_ref[...], 0.0)


@functools.cache
def _make_final_call(n, do, r):
    return pl.pallas_call(
        _final_body,
        grid=(n // r,),
        in_specs=[
            pl.BlockSpec((NC, r, do), lambda i: (0, i, 0)),
            pl.BlockSpec((r, 1), lambda i: (i, 0)),
            pl.BlockSpec((1, do), lambda i: (0, 0)),
        ],
        out_specs=pl.BlockSpec((r, do), lambda i: (i, 0)),
        out_shape=jax.ShapeDtypeStruct((n, do), jnp.float32),
        name="gcn_final_tc",
    )


def kernel(x, edge_index, W1, b1, W2, b2):
    n, d_in = x.shape
    e = edge_index.shape[1]
    d_hid = W1.shape[1]
    d_out = W2.shape[1]
    ew = e // NW
    nch = ew // K
    r = 1000

    ones1 = jnp.ones((128,), jnp.float32)
    rpt8 = (n // NS) & ~7
    nzr = rpt8 + (n - NS * rpt8)
    zeros1 = jnp.zeros((nzr,), jnp.float32)
    zeros_d = jnp.zeros((nzr, d_in), jnp.float32)

    edges = edge_index.reshape(-1)
    deg = _make_deg_kernel(n, ew, e)(edges, ones1, zeros1)
    deg4 = deg.reshape(NC * 2, n).T
    y1, rso, rsi = _make_scale_call(n, d_in, r)(deg4, x)
    p1 = _make_agg_kernel(n, d_in, ew, e)(y1, edges, zeros_d)
    y2 = _make_mid_call(n, d_in, d_hid, d_out, r)(
        p1, rso, rsi, W1, b1.reshape(1, -1), W2)
    p2 = _make_agg_kernel(n, d_out, ew, e)(y2, edges, zeros_d)
    h2 = _make_final_call(n, d_out, r)(p2, rsi, b2.reshape(1, -1))
    return h2


# speedup vs baseline: 11.7001x; 1.0617x over previous
import functools

import jax
import jax.numpy as jnp
from jax import lax
from jax.experimental import pallas as pl
from jax.experimental.pallas import tpu as pltpu
from jax.experimental.pallas import tpu_sc as plsc

NC = 2
NS = 16
NW = NC * NS
K = 80
NBUF = 2


def _mesh():
    return plsc.VectorSubcoreMesh(
        core_axis_name="c", subcore_axis_name="s", num_cores=NC, num_subcores=NS
    )


@functools.cache
def _make_deg_kernel(n, ew, eoff):
    rpt = (n // NS) & ~7
    tail = n - NS * rpt
    kd = 128
    nch = ew // kd
    kt = ew - nch * kd

    def body(edges_hbm, ones_hbm, zeros_hbm, out_hbm,
             si0, si1, di0, di1, si_t, di_t, ones_v, ones_t, stage,
             ssem0, ssem1, dsem0, dsem1,
             sa0, sa1, sb0, sb1, deg_o, deg_i):
        si = (si0, si1)
        di = (di0, di1)
        ssem = (ssem0, ssem1)
        dsem = (dsem0, dsem1)
        sa = (sa0, sa1)
        sb = (sb0, sb1)
        c = lax.axis_index("c")
        s = lax.axis_index("s")
        wid = c * NS + s
        base = pl.multiple_of(wid * ew, 8)
        pltpu.sync_copy(ones_hbm, ones_v)

        def load(j, b):
            off = base + pl.multiple_of(j * kd, 8)
            pltpu.async_copy(edges_hbm.at[pl.ds(off, kd)], si[b], ssem[b])
            pltpu.async_copy(edges_hbm.at[pl.ds(eoff + off, kd)], di[b], dsem[b])

        def wait_load(j, b):
            off = base + pl.multiple_of(j * kd, 8)
            pltpu.make_async_copy(edges_hbm.at[pl.ds(off, kd)], si[b], ssem[b]).wait()
            pltpu.make_async_copy(edges_hbm.at[pl.ds(eoff + off, kd)], di[b], dsem[b]).wait()

        def fire_scat(b):
            pltpu.async_copy(ones_v, deg_o.at[si[b]], sa[b], add=True)
            pltpu.async_copy(ones_v, deg_i.at[di[b]], sb[b], add=True)

        def drain_scat(b):
            pltpu.make_async_copy(ones_v, deg_o.at[si[b]], sa[b]).wait()
            pltpu.make_async_copy(ones_v, deg_i.at[di[b]], sb[b]).wait()

        load(0, 0)
        pltpu.sync_copy(zeros_hbm, stage)
        pltpu.sync_copy(stage.at[pl.ds(0, rpt)], deg_o.at[pl.ds(s * rpt, rpt)])
        pltpu.sync_copy(stage.at[pl.ds(0, rpt)], deg_i.at[pl.ds(s * rpt, rpt)])

        @pl.when(s == NS - 1)
        def _():
            pltpu.sync_copy(stage.at[pl.ds(0, tail)],
                            deg_o.at[pl.ds(NS * rpt, tail)])
            pltpu.sync_copy(stage.at[pl.ds(0, tail)],
                            deg_i.at[pl.ds(NS * rpt, tail)])

        plsc.subcore_barrier()
        wait_load(0, 0)
        fire_scat(0)
        load(1, 1)

        main = ((nch - 2) // 2) * 2
        if main >= 2:
            @pl.loop(1, main + 1, step=2)
            def _(jo):
                for uo in range(2):
                    j = jo + uo
                    u = (1 + uo) % 2
                    wait_load(j, u)
                    fire_scat(u)
                    drain_scat(1 - u)
                    load(j + 1, 1 - u)

        for j in range(main + 1, nch):
            u = j % 2
            wait_load(j, u)
            fire_scat(u)
            drain_scat(1 - u)
            if j + 1 < nch:
                load(j + 1, 1 - u)

        drain_scat((nch - 1) % 2)

        if kt:
            toff = base + pl.multiple_of(nch * kd, 8)
            pltpu.sync_copy(edges_hbm.at[pl.ds(toff, kt)], si_t)
            pltpu.sync_copy(edges_hbm.at[pl.ds(eoff + toff, kt)], di_t)
            pltpu.sync_copy(ones_hbm.at[pl.ds(0, kt)], ones_t)
            pltpu.sync_copy(ones_t, deg_o.at[si_t], add=True)
            pltpu.sync_copy(ones_t, deg_i.at[di_t], add=True)

        plsc.subcore_barrier()

        for t, ref in ((0, deg_o), (1, deg_i)):
            obase = pl.multiple_of((c * 2 + t) * n, 8)
            pltpu.sync_copy(ref.at[pl.ds(s * rpt, rpt)], stage.at[pl.ds(0, rpt)])
            pltpu.sync_copy(stage.at[pl.ds(0, rpt)],
                            out_hbm.at[pl.ds(obase + s * rpt, rpt)])

            @pl.when(s == NS - 1)
            def _():
                pltpu.sync_copy(ref.at[pl.ds(NS * rpt, tail)],
                                stage.at[pl.ds(rpt, tail)])
                pltpu.sync_copy(stage.at[pl.ds(rpt, tail)],
                                out_hbm.at[pl.ds(obase + NS * rpt, tail)])

    return pl.kernel(
        body,
        out_type=jax.ShapeDtypeStruct((NC * 2 * n,), jnp.float32),
        mesh=_mesh(),
        scratch_types=[
            pltpu.VMEM((kd,), jnp.int32),
            pltpu.VMEM((kd,), jnp.int32),
            pltpu.VMEM((kd,), jnp.int32),
            pltpu.VMEM((kd,), jnp.int32),
            pltpu.VMEM((max(kt, 8),), jnp.int32),
            pltpu.VMEM((max(kt, 8),), jnp.int32),
            pltpu.VMEM((kd,), jnp.float32),
            pltpu.VMEM((max(kt, 8),), jnp.float32),
            pltpu.VMEM((rpt + tail,), jnp.float32),
            pltpu.SemaphoreType.DMA,
            pltpu.SemaphoreType.DMA,
            pltpu.SemaphoreType.DMA,
            pltpu.SemaphoreType.DMA,
            pltpu.SemaphoreType.DMA,
            pltpu.SemaphoreType.DMA,
            pltpu.SemaphoreType.DMA,
            pltpu.SemaphoreType.DMA,
            pltpu.VMEM_SHARED((n,), jnp.float32),
            pltpu.VMEM_SHARED((n,), jnp.float32),
        ],
        name="gcn_degrees_sc",
    )


@functools.cache
def _make_agg_kernel(n, d, ew, eoff):
    rpt = (n // NS) & ~7
    tailr = n - NS * rpt
    nch = ew // K

    nb = 3

    def body(y_hbm, edges_hbm, zeros_hbm, out_hbm,
             si0, si1, si2, di0, di1, di2, rows,
             ss0, ss1, ss2, ds0, ds1, ds2, gs0, gs1, gs2, acc):
        si = (si0, si1, si2)
        di = (di0, di1, di2)
        ssem = (ss0, ss1, ss2)
        dsem = (ds0, ds1, ds2)
        gsem = (gs0, gs1, gs2)
        c = lax.axis_index("c")
        s = lax.axis_index("s")
        wid = c * NS + s
        base = pl.multiple_of(wid * ew, 8)

        def load(j, b):
            off = base + pl.multiple_of(j * K, K)
            pltpu.async_copy(edges_hbm.at[pl.ds(off, K)], si[b], ssem[b])
            pltpu.async_copy(edges_hbm.at[pl.ds(eoff + off, K)], di[b], dsem[b])

        def gather(j, b):
            off = base + pl.multiple_of(j * K, K)
            pltpu.make_async_copy(edges_hbm.at[pl.ds(off, K)], si[b], ssem[b]).wait()
            pltpu.async_copy(y_hbm.at[si[b]], rows.at[b], gsem[b])

        def scat(j, b):
            off = base + pl.multiple_of(j * K, K)
            pltpu.make_async_copy(y_hbm.at[si[b]], rows.at[b], gsem[b]).wait()
            pltpu.make_async_copy(edges_hbm.at[pl.ds(eoff + off, K)], di[b], dsem[b]).wait()
            pltpu.sync_copy(rows.at[b], acc.at[di[b]], add=True)

        load(0, 0)
        load(1, 1)
        gather(0, 0)
        pltpu.sync_copy(zeros_hbm.at[pl.ds(0, rpt)], acc.at[pl.ds(s * rpt, rpt)])

        @pl.when(s == NS - 1)
        def _():
            pltpu.sync_copy(zeros_hbm.at[pl.ds(0, tailr)],
                            acc.at[pl.ds(NS * rpt, tailr)])

        plsc.subcore_barrier()

        main = nch - (nch % nb) - nb
        if main > 0:
            @pl.loop(0, main, step=nb)
            def _(jo):
                for u in range(nb):
                    j = jo + u
                    load(j + 2, (u + 2) % nb)
                    gather(j + 1, (u + 1) % nb)
                    scat(j, u)

        for j in range(max(main, 0), nch):
            if j + 2 < nch:
                load(j + 2, (j + 2) % nb)
            if j + 1 < nch:
                gather(j + 1, (j + 1) % nb)
            scat(j, j % nb)

        plsc.subcore_barrier()
        pltpu.sync_copy(acc.at[pl.ds(s * rpt, rpt)],
                        out_hbm.at[c, pl.ds(s * rpt, rpt)])

        @pl.when(s == NS - 1)
        def _():
            pltpu.sync_copy(acc.at[pl.ds(NS * rpt, tailr)],
                            out_hbm.at[c, pl.ds(NS * rpt, tailr)])

    return pl.kernel(
        body,
        out_type=jax.ShapeDtypeStruct((NC, n, d), jnp.float32),
        mesh=_mesh(),
        scratch_types=[
            pltpu.VMEM((K,), jnp.int32),
            pltpu.VMEM((K,), jnp.int32),
            pltpu.VMEM((K,), jnp.int32),
            pltpu.VMEM((K,), jnp.int32),
            pltpu.VMEM((K,), jnp.int32),
            pltpu.VMEM((K,), jnp.int32),
            pltpu.VMEM((nb, K, d), jnp.float32),
            pltpu.SemaphoreType.DMA,
            pltpu.SemaphoreType.DMA,
            pltpu.SemaphoreType.DMA,
            pltpu.SemaphoreType.DMA,
            pltpu.SemaphoreType.DMA,
            pltpu.SemaphoreType.DMA,
            pltpu.SemaphoreType.DMA,
            pltpu.SemaphoreType.DMA,
            pltpu.SemaphoreType.DMA,
            pltpu.VMEM_SHARED((n, d), jnp.float32),
        ],
        name="gcn_edge_agg_sc",
    )


def _scale_body(deg_ref, x_ref, y_ref, rso_ref, rsi_ref):
    d = deg_ref[...]
    rso = lax.rsqrt(jnp.maximum(d[:, 0:1] + d[:, 2:3], 1.0))
    rsi = lax.rsqrt(jnp.maximum(d[:, 1:2] + d[:, 3:4], 1.0))
    rso_ref[...] = rso
    rsi_ref[...] = rsi
    y_ref[...] = x_ref[...] * rso


@functools.cache
def _make_scale_call(n, d, r):
    return pl.pallas_call(
        _scale_body,
        grid=(n // r,),
        in_specs=[
            pl.BlockSpec((r, 4), lambda i: (i, 0)),
            pl.BlockSpec((r, d), lambda i: (i, 0)),
        ],
        out_specs=[
            pl.BlockSpec((r, d), lambda i: (i, 0)),
            pl.BlockSpec((r, 1), lambda i: (i, 0)),
            pl.BlockSpec((r, 1), lambda i: (i, 0)),
        ],
        out_shape=[
            jax.ShapeDtypeStruct((n, d), jnp.float32),
            jax.ShapeDtypeStruct((n, 1), jnp.float32),
            jax.ShapeDtypeStruct((n, 1), jnp.float32),
        ],
        name="gcn_scale_tc",
    )


def _mid_body(p_ref, rso_ref, rsi_ref, w1_ref, b1_ref, w2_ref, y2_ref):
    agg = (p_ref[0] + p_ref[1]) * rsi_ref[...]
    h1 = jnp.dot(agg, w1_ref[...], preferred_element_type=jnp.float32,
                 precision=lax.Precision.DEFAULT) + b1_ref[...]
    h1 = jnp.maximum(h1, 0.0)
    y2_ref[...] = jnp.dot(h1 * rso_ref[...], w2_ref[...],
                          preferred_element_type=jnp.float32,
                          precision=lax.Precision.DEFAULT)


@functools.cache
def _make_mid_call(n, d, dh, do, r):
    return pl.pallas_call(
        _mid_body,
        grid=(n // r,),
        in_specs=[
            pl.BlockSpec((NC, r, d), lambda i: (0, i, 0)),
            pl.BlockSpec((r, 1), lambda i: (i, 0)),
            pl.BlockSpec((r, 1), lambda i: (i, 0)),
            pl.BlockSpec((d, dh), lambda i: (0, 0)),
            pl.BlockSpec((1, dh), lambda i: (0, 0)),
            pl.BlockSpec((dh, do), lambda i: (0, 0)),
        ],
        out_specs=pl.BlockSpec((r, do), lambda i: (i, 0)),
        out_shape=jax.ShapeDtypeStruct((n, do), jnp.float32),
        name="gcn_mid_tc",
    )


def _final_body(p_ref, rsi_ref, b2_ref, h2_ref):
    agg = (p_ref[0] + p_ref[1]) * rsi_ref[...]
    h2_ref[...] = jnp.maximum(agg + b2_ref[...], 0.0)


@functools.cache
def _make_final_call(n, do, r):
    return pl.pallas_call(
        _final_body,
        grid=(n // r,),
        in_specs=[
            pl.BlockSpec((NC, r, do), lambda i: (0, i, 0)),
            pl.BlockSpec((r, 1), lambda i: (i, 0)),
            pl.BlockSpec((1, do), lambda i: (0, 0)),
        ],
        out_specs=pl.BlockSpec((r, do), lambda i: (i, 0)),
        out_shape=jax.ShapeDtypeStruct((n, do), jnp.float32),
        name="gcn_final_tc",
    )


def kernel(x, edge_index, W1, b1, W2, b2):
    n, d_in = x.shape
    e = edge_index.shape[1]
    d_hid = W1.shape[1]
    d_out = W2.shape[1]
    ew = e // NW
    nch = ew // K
    r = 1000

    ones1 = jnp.ones((128,), jnp.float32)
    rpt8 = (n // NS) & ~7
    nzr = rpt8 + (n - NS * rpt8)
    zeros1 = jnp.zeros((nzr,), jnp.float32)
    zeros_d = jnp.zeros((nzr, d_in), jnp.float32)

    edges = edge_index.reshape(-1)
    deg = _make_deg_kernel(n, ew, e)(edges, ones1, zeros1)
    deg4 = deg.reshape(NC * 2, n).T
    y1, rso, rsi = _make_scale_call(n, d_in, r)(deg4, x)
    p1 = _make_agg_kernel(n, d_in, ew, e)(y1, edges, zeros_d)
    y2 = _make_mid_call(n, d_in, d_hid, d_out, r)(
        p1, rso, rsi, W1, b1.reshape(1, -1), W2)
    p2 = _make_agg_kernel(n, d_out, ew, e)(y2, edges, zeros_d)
    h2 = _make_final_call(n, d_out, r)(p2, rsi, b2.reshape(1, -1))
    return h2


# speedup vs baseline: 11.7053x; 1.0004x over previous
import functools

import jax
import jax.numpy as jnp
from jax import lax
from jax.experimental import pallas as pl
from jax.experimental.pallas import tpu as pltpu
from jax.experimental.pallas import tpu_sc as plsc

NC = 2
NS = 16
NW = NC * NS
K = 80
NBUF = 2


def _mesh():
    return plsc.VectorSubcoreMesh(
        core_axis_name="c", subcore_axis_name="s", num_cores=NC, num_subcores=NS
    )


@functools.cache
def _make_deg_kernel(n, ew, eoff):
    rpt = (n // NS) & ~7
    tail = n - NS * rpt
    kd = 128
    nch = ew // kd
    kt = ew - nch * kd

    def body(edges_hbm, ones_hbm, zeros_hbm, out_hbm,
             si0, si1, di0, di1, si_t, di_t, ones_v, ones_t, stage,
             ssem0, ssem1, dsem0, dsem1,
             sa0, sa1, sb0, sb1, deg_o, deg_i):
        si = (si0, si1)
        di = (di0, di1)
        ssem = (ssem0, ssem1)
        dsem = (dsem0, dsem1)
        sa = (sa0, sa1)
        sb = (sb0, sb1)
        c = lax.axis_index("c")
        s = lax.axis_index("s")
        wid = c * NS + s
        base = pl.multiple_of(wid * ew, 8)
        pltpu.sync_copy(ones_hbm, ones_v)

        def load(j, b):
            off = base + pl.multiple_of(j * kd, 8)
            pltpu.async_copy(edges_hbm.at[pl.ds(off, kd)], si[b], ssem[b])
            pltpu.async_copy(edges_hbm.at[pl.ds(eoff + off, kd)], di[b], dsem[b])

        def wait_load(j, b):
            off = base + pl.multiple_of(j * kd, 8)
            pltpu.make_async_copy(edges_hbm.at[pl.ds(off, kd)], si[b], ssem[b]).wait()
            pltpu.make_async_copy(edges_hbm.at[pl.ds(eoff + off, kd)], di[b], dsem[b]).wait()

        def fire_scat(b):
            pltpu.async_copy(ones_v, deg_o.at[si[b]], sa[b], add=True)
            pltpu.async_copy(ones_v, deg_i.at[di[b]], sb[b], add=True)

        def drain_scat(b):
            pltpu.make_async_copy(ones_v, deg_o.at[si[b]], sa[b]).wait()
            pltpu.make_async_copy(ones_v, deg_i.at[di[b]], sb[b]).wait()

        load(0, 0)
        pltpu.sync_copy(zeros_hbm, stage)
        pltpu.sync_copy(stage.at[pl.ds(0, rpt)], deg_o.at[pl.ds(s * rpt, rpt)])
        pltpu.sync_copy(stage.at[pl.ds(0, rpt)], deg_i.at[pl.ds(s * rpt, rpt)])

        @pl.when(s == NS - 1)
        def _():
            pltpu.sync_copy(stage.at[pl.ds(0, tail)],
                            deg_o.at[pl.ds(NS * rpt, tail)])
            pltpu.sync_copy(stage.at[pl.ds(0, tail)],
                            deg_i.at[pl.ds(NS * rpt, tail)])

        plsc.subcore_barrier()
        wait_load(0, 0)
        fire_scat(0)
        load(1, 1)

        main = ((nch - 2) // 2) * 2
        if main >= 2:
            @pl.loop(1, main + 1, step=2)
            def _(jo):
                for uo in range(2):
                    j = jo + uo
                    u = (1 + uo) % 2
                    wait_load(j, u)
                    fire_scat(u)
                    drain_scat(1 - u)
                    load(j + 1, 1 - u)

        for j in range(main + 1, nch):
            u = j % 2
            wait_load(j, u)
            fire_scat(u)
            drain_scat(1 - u)
            if j + 1 < nch:
                load(j + 1, 1 - u)

        drain_scat((nch - 1) % 2)

        if kt:
            toff = base + pl.multiple_of(nch * kd, 8)
            pltpu.sync_copy(edges_hbm.at[pl.ds(toff, kt)], si_t)
            pltpu.sync_copy(edges_hbm.at[pl.ds(eoff + toff, kt)], di_t)
            pltpu.sync_copy(ones_hbm.at[pl.ds(0, kt)], ones_t)
            pltpu.sync_copy(ones_t, deg_o.at[si_t], add=True)
            pltpu.sync_copy(ones_t, deg_i.at[di_t], add=True)

        plsc.subcore_barrier()

        for t, ref in ((0, deg_o), (1, deg_i)):
            obase = pl.multiple_of((c * 2 + t) * n, 8)
            pltpu.sync_copy(ref.at[pl.ds(s * rpt, rpt)], stage.at[pl.ds(0, rpt)])
            pltpu.sync_copy(stage.at[pl.ds(0, rpt)],
                            out_hbm.at[pl.ds(obase + s * rpt, rpt)])

            @pl.when(s == NS - 1)
            def _():
                pltpu.sync_copy(ref.at[pl.ds(NS * rpt, tail)],
                                stage.at[pl.ds(rpt, tail)])
                pltpu.sync_copy(stage.at[pl.ds(rpt, tail)],
                                out_hbm.at[pl.ds(obase + NS * rpt, tail)])

    return pl.kernel(
        body,
        out_type=jax.ShapeDtypeStruct((NC * 2 * n,), jnp.float32),
        mesh=_mesh(),
        scratch_types=[
            pltpu.VMEM((kd,), jnp.int32),
            pltpu.VMEM((kd,), jnp.int32),
            pltpu.VMEM((kd,), jnp.int32),
            pltpu.VMEM((kd,), jnp.int32),
            pltpu.VMEM((max(kt, 8),), jnp.int32),
            pltpu.VMEM((max(kt, 8),), jnp.int32),
            pltpu.VMEM((kd,), jnp.float32),
            pltpu.VMEM((max(kt, 8),), jnp.float32),
            pltpu.VMEM((rpt + tail,), jnp.float32),
            pltpu.SemaphoreType.DMA,
            pltpu.SemaphoreType.DMA,
            pltpu.SemaphoreType.DMA,
            pltpu.SemaphoreType.DMA,
            pltpu.SemaphoreType.DMA,
            pltpu.SemaphoreType.DMA,
            pltpu.SemaphoreType.DMA,
            pltpu.SemaphoreType.DMA,
            pltpu.VMEM_SHARED((n,), jnp.float32),
            pltpu.VMEM_SHARED((n,), jnp.float32),
        ],
        name="gcn_degrees_sc",
    )


@functools.cache
def _make_agg_kernel(n, d, ew, eoff):
    rpt = (n // NS) & ~7
    tailr = n - NS * rpt
    nch = ew // K

    nb = 3

    def body(y_hbm, edges_hbm, zeros_hbm, out_hbm,
             si0, si1, si2, di0, di1, di2, rows,
             ss0, ss1, ss2, ds0, ds1, ds2, gs0, gs1, gs2,
             cs0, cs1, cs2, acc):
        si = (si0, si1, si2)
        di = (di0, di1, di2)
        ssem = (ss0, ss1, ss2)
        dsem = (ds0, ds1, ds2)
        gsem = (gs0, gs1, gs2)
        csem = (cs0, cs1, cs2)
        c = lax.axis_index("c")
        s = lax.axis_index("s")
        wid = c * NS + s
        base = pl.multiple_of(wid * ew, 8)

        def load(j, b):
            off = base + pl.multiple_of(j * K, K)
            pltpu.async_copy(edges_hbm.at[pl.ds(off, K)], si[b], ssem[b])
            pltpu.async_copy(edges_hbm.at[pl.ds(eoff + off, K)], di[b], dsem[b])

        def gather(j, b):
            off = base + pl.multiple_of(j * K, K)
            pltpu.make_async_copy(edges_hbm.at[pl.ds(off, K)], si[b], ssem[b]).wait()
            pltpu.async_copy(y_hbm.at[si[b]], rows.at[b], gsem[b])

        def fire_scat(j, b):
            off = base + pl.multiple_of(j * K, K)
            pltpu.make_async_copy(y_hbm.at[si[b]], rows.at[b], gsem[b]).wait()
            pltpu.make_async_copy(edges_hbm.at[pl.ds(eoff + off, K)], di[b], dsem[b]).wait()
            pltpu.async_copy(rows.at[b], acc.at[di[b]], csem[b], add=True)

        def drain_scat(b):
            pltpu.make_async_copy(rows.at[b], acc.at[di[b]], csem[b]).wait()

        load(0, 0)
        load(1, 1)
        gather(0, 0)
        pltpu.sync_copy(zeros_hbm.at[pl.ds(0, rpt)], acc.at[pl.ds(s * rpt, rpt)])

        @pl.when(s == NS - 1)
        def _():
            pltpu.sync_copy(zeros_hbm.at[pl.ds(0, tailr)],
                            acc.at[pl.ds(NS * rpt, tailr)])

        plsc.subcore_barrier()

        load(2, 2)
        gather(1, 1)
        fire_scat(0, 0)

        ntrip = max((nch - 3) // nb, 0)
        if ntrip > 0:
            @pl.loop(1, 1 + ntrip * nb, step=nb)
            def _(jo):
                for uo in range(nb):
                    j = jo + uo
                    u = (1 + uo) % nb
                    drain_scat((u + 2) % nb)
                    load(j + 2, (u + 2) % nb)
                    gather(j + 1, (u + 1) % nb)
                    fire_scat(j, u)

        for j in range(1 + ntrip * nb, nch):
            u = j % nb
            drain_scat((u + 2) % nb)
            if j + 2 < nch:
                load(j + 2, (u + 2) % nb)
            if j + 1 < nch:
                gather(j + 1, (u + 1) % nb)
            fire_scat(j, u)

        drain_scat((nch - 1) % nb)
        plsc.subcore_barrier()
        pltpu.sync_copy(acc.at[pl.ds(s * rpt, rpt)],
                        out_hbm.at[c, pl.ds(s * rpt, rpt)])

        @pl.when(s == NS - 1)
        def _():
            pltpu.sync_copy(acc.at[pl.ds(NS * rpt, tailr)],
                            out_hbm.at[c, pl.ds(NS * rpt, tailr)])

    return pl.kernel(
        body,
        out_type=jax.ShapeDtypeStruct((NC, n, d), jnp.float32),
        mesh=_mesh(),
        scratch_types=[
            pltpu.VMEM((K,), jnp.int32),
            pltpu.VMEM((K,), jnp.int32),
            pltpu.VMEM((K,), jnp.int32),
            pltpu.VMEM((K,), jnp.int32),
            pltpu.VMEM((K,), jnp.int32),
            pltpu.VMEM((K,), jnp.int32),
            pltpu.VMEM((nb, K, d), jnp.float32),
            pltpu.SemaphoreType.DMA,
            pltpu.SemaphoreType.DMA,
            pltpu.SemaphoreType.DMA,
            pltpu.SemaphoreType.DMA,
            pltpu.SemaphoreType.DMA,
            pltpu.SemaphoreType.DMA,
            pltpu.SemaphoreType.DMA,
            pltpu.SemaphoreType.DMA,
            pltpu.SemaphoreType.DMA,
            pltpu.SemaphoreType.DMA,
            pltpu.SemaphoreType.DMA,
            pltpu.SemaphoreType.DMA,
            pltpu.VMEM_SHARED((n, d), jnp.float32),
        ],
        name="gcn_edge_agg_sc",
    )


def _scale_body(deg_ref, x_ref, y_ref, rso_ref, rsi_ref):
    d = deg_ref[...]
    rso = lax.rsqrt(jnp.maximum(d[:, 0:1] + d[:, 2:3], 1.0))
    rsi = lax.rsqrt(jnp.maximum(d[:, 1:2] + d[:, 3:4], 1.0))
    rso_ref[...] = rso
    rsi_ref[...] = rsi
    y_ref[...] = x_ref[...] * rso


@functools.cache
def _make_scale_call(n, d, r):
    return pl.pallas_call(
        _scale_body,
        grid=(n // r,),
        in_specs=[
            pl.BlockSpec((r, 4), lambda i: (i, 0)),
            pl.BlockSpec((r, d), lambda i: (i, 0)),
        ],
        out_specs=[
            pl.BlockSpec((r, d), lambda i: (i, 0)),
            pl.BlockSpec((r, 1), lambda i: (i, 0)),
            pl.BlockSpec((r, 1), lambda i: (i, 0)),
        ],
        out_shape=[
            jax.ShapeDtypeStruct((n, d), jnp.float32),
            jax.ShapeDtypeStruct((n, 1), jnp.float32),
            jax.ShapeDtypeStruct((n, 1), jnp.float32),
        ],
        name="gcn_scale_tc",
    )


def _mid_body(p_ref, rso_ref, rsi_ref, w1_ref, b1_ref, w2_ref, y2_ref):
    agg = (p_ref[0] + p_ref[1]) * rsi_ref[...]
    h1 = jnp.dot(agg, w1_ref[...], preferred_element_type=jnp.float32,
                 precision=lax.Precision.DEFAULT) + b1_ref[...]
    h1 = jnp.maximum(h1, 0.0)
    y2_ref[...] = jnp.dot(h1 * rso_ref[...], w2_ref[...],
                          preferred_element_type=jnp.float32,
                          precision=lax.Precision.DEFAULT)


@functools.cache
def _make_mid_call(n, d, dh, do, r):
    return pl.pallas_call(
        _mid_body,
        grid=(n // r,),
        in_specs=[
            pl.BlockSpec((NC, r, d), lambda i: (0, i, 0)),
            pl.BlockSpec((r, 1), lambda i: (i, 0)),
            pl.BlockSpec((r, 1), lambda i: (i, 0)),
            pl.BlockSpec((d, dh), lambda i: (0, 0)),
            pl.BlockSpec((1, dh), lambda i: (0, 0)),
            pl.BlockSpec((dh, do), lambda i: (0, 0)),
        ],
        out_specs=pl.BlockSpec((r, do), lambda i: (i, 0)),
        out_shape=jax.ShapeDtypeStruct((n, do), jnp.float32),
        name="gcn_mid_tc",
    )


def _final_body(p_ref, rsi_ref, b2_ref, h2_ref):
    agg = (p_ref[0] + p_ref[1]) * rsi_ref[...]
    h2_ref[...] = jnp.maximum(agg + b2_ref[...], 0.0)


@functools.cache
def _make_final_call(n, do, r):
    return pl.pallas_call(
        _final_body,
        grid=(n // r,),
        in_specs=[
            pl.BlockSpec((NC, r, do), lambda i: (0, i, 0)),
            pl.BlockSpec((r, 1), lambda i: (i, 0)),
            pl.BlockSpec((1, do), lambda i: (0, 0)),
        ],
        out_specs=pl.BlockSpec((r, do), lambda i: (i, 0)),
        out_shape=jax.ShapeDtypeStruct((n, do), jnp.float32),
        name="gcn_final_tc",
    )


def kernel(x, edge_index, W1, b1, W2, b2):
    n, d_in = x.shape
    e = edge_index.shape[1]
    d_hid = W1.shape[1]
    d_out = W2.shape[1]
    ew = e // NW
    nch = ew // K
    r = 1000

    ones1 = jnp.ones((128,), jnp.float32)
    rpt8 = (n // NS) & ~7
    nzr = rpt8 + (n - NS * rpt8)
    zeros1 = jnp.zeros((nzr,), jnp.float32)
    zeros_d = jnp.zeros((nzr, d_in), jnp.float32)

    edges = edge_index.reshape(-1)
    deg = _make_deg_kernel(n, ew, e)(edges, ones1, zeros1)
    deg4 = deg.reshape(NC * 2, n).T
    y1, rso, rsi = _make_scale_call(n, d_in, r)(deg4, x)
    p1 = _make_agg_kernel(n, d_in, ew, e)(y1, edges, zeros_d)
    y2 = _make_mid_call(n, d_in, d_hid, d_out, r)(
        p1, rso, rsi, W1, b1.reshape(1, -1), W2)
    p2 = _make_agg_kernel(n, d_out, ew, e)(y2, edges, zeros_d)
    h2 = _make_final_call(n, d_out, r)(p2, rsi, b2.reshape(1, -1))
    return h2
